# baseline (speedup 1.0000x reference)
import functools

import jax
import jax.numpy as jnp
from jax import lax
from jax.experimental import pallas as pl
from jax.experimental.pallas import tpu as pltpu
from jax.experimental.pallas import tpu_sc as plsc

B, N, C_IN, M, K, OUT = 8, 8192, 64, 2048, 32, 128
D_CAT = 3 + C_IN
EPS = 1e-5
BM = 128
PW = 8
TW = 80
IDX_MASK = 8191
NW = 32
GCH = 16


def _topk_body(xyzp_ref, ct_ref, out_ref, res_ref):
    x = xyzp_ref[...]
    c = ct_ref[...]
    xn2 = jnp.sum(x * x, axis=1, keepdims=True)
    cn2 = jnp.sum(c * c, axis=0, keepdims=True)
    dot = jnp.dot(x, c, preferred_element_type=jnp.float32)
    d2 = jnp.maximum(xn2 + cn2 - 2.0 * dot, 0.0)
    bits = lax.bitcast_convert_type(d2, jnp.int32)
    rows = lax.broadcasted_iota(jnp.int32, (N, BM), 0)
    keys = (bits & jnp.int32(~IDX_MASK)) | rows
    maxi = jnp.int32(0x7FFFFFFF)

    kr = keys.reshape(32, 32, 8, BM)
    m1 = jnp.full((32, 8, BM), maxi)
    m2, m3, m4 = m1, m1, m1
    for step in range(32):
        v = kr[:, step]
        m1, v = jnp.minimum(m1, v), jnp.maximum(m1, v)
        m2, v = jnp.minimum(m2, v), jnp.maximum(m2, v)
        m3, v = jnp.minimum(m3, v), jnp.maximum(m3, v)
        m4 = jnp.minimum(m4, v)
    cand = jnp.stack([m1, m2, m3, m4]).reshape(4 * 32 * 8, BM)

    def sbody(k, prev):
        m = jnp.min(jnp.where(cand > prev, cand, maxi), axis=0)
        res_ref[pl.ds(k, 1), :] = m.reshape(1, BM)
        return m

    mlast = lax.fori_loop(0, K, sbody, jnp.full((BM,), -1, jnp.int32))

    count = jnp.sum((keys <= mlast).astype(jnp.int32), axis=0)
    ok = jnp.all(count == K)

    @pl.when(ok)
    def _():
        out_ref[...] = res_ref[...] & IDX_MASK

    @pl.when(jnp.logical_not(ok))
    def _():
        def fbody(k, prev):
            m = jnp.min(jnp.where(keys > prev, keys, maxi), axis=0)
            out_ref[pl.ds(k, 1), :] = (m & IDX_MASK).reshape(1, BM)
            return m

        lax.fori_loop(0, K, fbody, jnp.full((BM,), -1, jnp.int32))


def _topk_b(xyzp_b, centers_t_b):
    return pl.pallas_call(
        _topk_body,
        grid=(M // BM,),
        in_specs=[
            pl.BlockSpec((N, PW), lambda mb: (0, 0)),
            pl.BlockSpec((PW, BM), lambda mb: (0, mb)),
        ],
        out_specs=pl.BlockSpec((K, BM), lambda mb: (0, mb)),
        out_shape=jax.ShapeDtypeStruct((K, M), jnp.int32),
        scratch_shapes=[pltpu.VMEM((K, BM), jnp.int32)],
    )(xyzp_b, centers_t_b)


def _gather_b(table, idx2):
    tot = K * M

    @functools.partial(
        pl.kernel,
        out_type=jax.ShapeDtypeStruct((tot, TW), jnp.float32),
        mesh=plsc.VectorSubcoreMesh(core_axis_name="c", subcore_axis_name="s"),
        compiler_params=pltpu.CompilerParams(use_tc_tiling_on_sc=False),
        scratch_types=[
            pltpu.VMEM((GCH, 128), jnp.int32),
            pltpu.VMEM((2, 128, TW), jnp.float32),
            pltpu.SemaphoreType.DMA,
            pltpu.SemaphoreType.DMA,
        ],
    )
    def k(table_hbm, idx_hbm, out_hbm, idx_v, rows_v, sem0, sem1):
        wid = lax.axis_index("s") * 2 + lax.axis_index("c")
        pltpu.sync_copy(idx_hbm.at[pl.ds(wid * GCH, GCH), :], idx_v)
        base = wid * GCH * 128
        sems = (sem0, sem1)
        pend = pltpu.async_copy(table_hbm.at[idx_v.at[0]], rows_v.at[0], sems[0])
        for j in range(GCH):
            nxt = None
            if j + 1 < GCH:
                nxt = pltpu.async_copy(
                    table_hbm.at[idx_v.at[j + 1]], rows_v.at[(j + 1) % 2],
                    sems[(j + 1) % 2])
            pend.wait()
            pltpu.sync_copy(rows_v.at[j % 2],
                            out_hbm.at[pl.ds(base + j * 128, 128), :])
            pend = nxt

    return k(table, idx2)


def _mlp_body(g_ref, cpad_ref, wf_ref, bf_ref, wa1_ref, wa2_ref, ba_ref, out_ref):
    g = g_ref[...]
    g2 = g.reshape(K * BM, TW)
    cpad = cpad_ref[...]
    wf = wf_ref[...]
    wa1 = wa1_ref[...]
    wa2 = wa2_ref[...]
    bf = bf_ref[...]
    ba = ba_ref[...]

    raw_f = jnp.dot(g2, wf, preferred_element_type=jnp.float32)
    corr_f = jnp.dot(cpad, wf[0:PW, :], preferred_element_type=jnp.float32)
    f3 = jnp.maximum(raw_f.reshape(K, BM, OUT) - corr_f[None] + bf[None], 0.0)
    fmean = jnp.mean(f3, axis=0)

    raw_a = jnp.dot(g2, wa1, preferred_element_type=jnp.float32)
    corr_a = jnp.dot(cpad, wa1[0:PW, :], preferred_element_type=jnp.float32)
    t = jnp.dot((f3 - fmean[None]).reshape(K * BM, OUT), wa2,
                preferred_element_type=jnp.float32)
    alpha = jnp.maximum(
        raw_a.reshape(K, BM, OUT) - corr_a[None] + t.reshape(K, BM, OUT) + ba[None],
        0.0)

    amax = jnp.max(alpha, axis=0)
    e = jnp.exp(alpha - amax[None])
    s = jnp.sum(e, axis=0)
    fr = jnp.sum(e * f3, axis=0) / s
    out_ref[...] = fr.T


def _mlp_b(g_b, cpad_b, wf, bf, wa1, wa2, ba):
    return pl.pallas_call(
        _mlp_body,
        grid=(M // BM,),
        in_specs=[
            pl.BlockSpec((K, BM, TW), lambda mb: (0, mb, 0)),
            pl.BlockSpec((BM, PW), lambda mb: (mb, 0)),
            pl.BlockSpec((TW, OUT), lambda mb: (0, 0)),
            pl.BlockSpec((1, OUT), lambda mb: (0, 0)),
            pl.BlockSpec((TW, OUT), lambda mb: (0, 0)),
            pl.BlockSpec((OUT, OUT), lambda mb: (0, 0)),
            pl.BlockSpec((1, OUT), lambda mb: (0, 0)),
        ],
        out_specs=pl.BlockSpec((OUT, BM), lambda mb: (0, mb)),
        out_shape=jax.ShapeDtypeStruct((OUT, M), jnp.float32),
    )(g_b, cpad_b, wf, bf, wa1, wa2, ba)


def kernel(xyz, feat_in, Wf, bf, gf, betaf, Wa, ba, ga, betaa):
    idx_center = jnp.linspace(0.0, N - 1, M).astype(jnp.int32)
    centers = xyz[:, idx_center, :]

    xyzp = jnp.concatenate(
        [xyz, jnp.zeros((B, N, PW - 3), jnp.float32)], axis=-1)
    cpad = xyzp[:, idx_center, :]
    centers_t = jnp.transpose(cpad, (0, 2, 1))

    feat_t = jnp.transpose(feat_in, (0, 2, 1))
    table = jnp.concatenate(
        [xyz, feat_t, jnp.zeros((B, N, TW - D_CAT), jnp.float32)],
        axis=-1).reshape(B * N, TW)

    scale_f = gf / jnp.sqrt(1.0 + EPS)
    scale_a = ga / jnp.sqrt(1.0 + EPS)
    wf_eff = Wf * scale_f[:, None]
    bf_eff = (bf * scale_f + betaf).reshape(1, OUT)
    wa_eff = Wa * scale_a[:, None]
    ba_eff = (ba * scale_a + betaa).reshape(1, OUT)

    wf_pad = jnp.zeros((TW, OUT), jnp.float32).at[:D_CAT].set(wf_eff.T)
    wa1_pad = jnp.zeros((TW, OUT), jnp.float32).at[:D_CAT].set(wa_eff[:, :D_CAT].T)
    wa2 = wa_eff[:, D_CAT:].T

    f_outs = []
    for b in range(B):
        idxl = _topk_b(xyzp[b], centers_t[b])
        gidx = (idxl + b * N).reshape(NW * GCH, 128)
        g_b = _gather_b(table, gidx).reshape(K, M, TW)
        f_outs.append(_mlp_b(g_b, cpad[b], wf_pad, bf_eff, wa1_pad, wa2, ba_eff))

    f_region = jnp.stack(f_outs)
    return centers, f_region


# speedup vs baseline: 1.0694x; 1.0694x over previous
import functools

import jax
import jax.numpy as jnp
from jax import lax
from jax.experimental import pallas as pl
from jax.experimental.pallas import tpu as pltpu
from jax.experimental.pallas import tpu_sc as plsc

B, N, C_IN, M, K, OUT = 8, 8192, 64, 2048, 32, 128
D_CAT = 3 + C_IN
EPS = 1e-5
BM = 128
BMT = 256
PW = 8
TW = 80
IDX_MASK = 8191
NW = 32
GCH = 16


def _topk_body(xyzp_ref, ct_ref, out_ref, res_ref):
    x = xyzp_ref[...]
    c = ct_ref[...]
    xn2 = jnp.sum(x * x, axis=1, keepdims=True)
    cn2 = jnp.sum(c * c, axis=0, keepdims=True)
    dot = jnp.dot(x, c, preferred_element_type=jnp.float32)
    d2 = jnp.maximum(xn2 + cn2 - 2.0 * dot, 0.0)
    bits = lax.bitcast_convert_type(d2, jnp.int32)
    rows = lax.broadcasted_iota(jnp.int32, (N, BMT), 0)
    keys = (bits & jnp.int32(~IDX_MASK)) | rows
    maxi = jnp.int32(0x7FFFFFFF)

    kr = keys.reshape(32, 32, 8, BMT)
    m1 = jnp.full((32, 8, BMT), maxi)
    m2, m3, m4 = m1, m1, m1
    for step in range(32):
        v = kr[:, step]
        m1, v = jnp.minimum(m1, v), jnp.maximum(m1, v)
        m2, v = jnp.minimum(m2, v), jnp.maximum(m2, v)
        m3, v = jnp.minimum(m3, v), jnp.maximum(m3, v)
        m4 = jnp.minimum(m4, v)
    cand = jnp.stack([m1, m2, m3, m4]).reshape(4 * 32 * 8, BMT)

    def sbody(k, prev):
        m = jnp.min(jnp.where(cand > prev, cand, maxi), axis=0)
        res_ref[pl.ds(k, 1), :] = m.reshape(1, BMT)
        return m

    mlast = lax.fori_loop(0, K, sbody, jnp.full((BMT,), -1, jnp.int32))

    count = jnp.sum((keys <= mlast).astype(jnp.int32), axis=0)
    ok = jnp.all(count == K)

    @pl.when(ok)
    def _():
        out_ref[...] = res_ref[...] & IDX_MASK

    @pl.when(jnp.logical_not(ok))
    def _():
        def fbody(k, prev):
            m = jnp.min(jnp.where(keys > prev, keys, maxi), axis=0)
            out_ref[pl.ds(k, 1), :] = (m & IDX_MASK).reshape(1, BMT)
            return m

        lax.fori_loop(0, K, fbody, jnp.full((BMT,), -1, jnp.int32))


def _topk_b(xyzp_b, centers_t_b):
    return pl.pallas_call(
        _topk_body,
        grid=(M // BMT,),
        in_specs=[
            pl.BlockSpec((N, PW), lambda mb: (0, 0)),
            pl.BlockSpec((PW, BMT), lambda mb: (0, mb)),
        ],
        out_specs=pl.BlockSpec((K, BMT), lambda mb: (0, mb)),
        out_shape=jax.ShapeDtypeStruct((K, M), jnp.int32),
        scratch_shapes=[pltpu.VMEM((K, BMT), jnp.int32)],
    )(xyzp_b, centers_t_b)


def _gather_b(table, idx2):
    tot = K * M

    @functools.partial(
        pl.kernel,
        out_type=jax.ShapeDtypeStruct((tot, TW), jnp.float32),
        mesh=plsc.VectorSubcoreMesh(core_axis_name="c", subcore_axis_name="s"),
        compiler_params=pltpu.CompilerParams(use_tc_tiling_on_sc=False),
        scratch_types=[
            pltpu.VMEM((GCH, 128), jnp.int32),
            pltpu.VMEM((2, 128, TW), jnp.float32),
            pltpu.SemaphoreType.DMA,
            pltpu.SemaphoreType.DMA,
        ],
    )
    def k(table_hbm, idx_hbm, out_hbm, idx_v, rows_v, sem0, sem1):
        wid = lax.axis_index("s") * 2 + lax.axis_index("c")
        pltpu.sync_copy(idx_hbm.at[pl.ds(wid * GCH, GCH), :], idx_v)
        base = wid * GCH * 128
        sems = (sem0, sem1)
        pend = pltpu.async_copy(table_hbm.at[idx_v.at[0]], rows_v.at[0], sems[0])
        for j in range(GCH):
            nxt = None
            if j + 1 < GCH:
                nxt = pltpu.async_copy(
                    table_hbm.at[idx_v.at[j + 1]], rows_v.at[(j + 1) % 2],
                    sems[(j + 1) % 2])
            pend.wait()
            pltpu.sync_copy(rows_v.at[j % 2],
                            out_hbm.at[pl.ds(base + j * 128, 128), :])
            pend = nxt

    return k(table, idx2)


def _mlp_body(g_ref, cpad_ref, wf_ref, bf_ref, wa1_ref, wa2_ref, ba_ref, out_ref):
    g = g_ref[...]
    g2 = g.reshape(K * BM, TW)
    cpad = cpad_ref[...]
    wf = wf_ref[...]
    wa1 = wa1_ref[...]
    wa2 = wa2_ref[...]
    bf = bf_ref[...]
    ba = ba_ref[...]

    raw_f = jnp.dot(g2, wf, preferred_element_type=jnp.float32)
    corr_f = jnp.dot(cpad, wf[0:PW, :], preferred_element_type=jnp.float32)
    f3 = jnp.maximum(raw_f.reshape(K, BM, OUT) - corr_f[None] + bf[None], 0.0)
    fmean = jnp.mean(f3, axis=0)

    raw_a = jnp.dot(g2, wa1, preferred_element_type=jnp.float32)
    corr_a = jnp.dot(cpad, wa1[0:PW, :], preferred_element_type=jnp.float32)
    t = jnp.dot((f3 - fmean[None]).reshape(K * BM, OUT), wa2,
                preferred_element_type=jnp.float32)
    alpha = jnp.maximum(
        raw_a.reshape(K, BM, OUT) - corr_a[None] + t.reshape(K, BM, OUT) + ba[None],
        0.0)

    amax = jnp.max(alpha, axis=0)
    e = jnp.exp(alpha - amax[None])
    s = jnp.sum(e, axis=0)
    fr = jnp.sum(e * f3, axis=0) / s
    out_ref[...] = fr


def _mlp_b(g_b, cpad_b, wf, bf, wa1, wa2, ba):
    return pl.pallas_call(
        _mlp_body,
        grid=(M // BM,),
        in_specs=[
            pl.BlockSpec((K, BM, TW), lambda mb: (0, mb, 0)),
            pl.BlockSpec((BM, PW), lambda mb: (mb, 0)),
            pl.BlockSpec((TW, OUT), lambda mb: (0, 0)),
            pl.BlockSpec((1, OUT), lambda mb: (0, 0)),
            pl.BlockSpec((TW, OUT), lambda mb: (0, 0)),
            pl.BlockSpec((OUT, OUT), lambda mb: (0, 0)),
            pl.BlockSpec((1, OUT), lambda mb: (0, 0)),
        ],
        out_specs=pl.BlockSpec((BM, OUT), lambda mb: (mb, 0)),
        out_shape=jax.ShapeDtypeStruct((M, OUT), jnp.float32),
    )(g_b, cpad_b, wf, bf, wa1, wa2, ba)


def kernel(xyz, feat_in, Wf, bf, gf, betaf, Wa, ba, ga, betaa):
    idx_center = jnp.linspace(0.0, N - 1, M).astype(jnp.int32)
    centers = xyz[:, idx_center, :]

    xyzp = jnp.concatenate(
        [xyz, jnp.zeros((B, N, PW - 3), jnp.float32)], axis=-1)
    cpad = xyzp[:, idx_center, :]
    centers_t = jnp.transpose(cpad, (0, 2, 1))

    feat_t = jnp.transpose(feat_in, (0, 2, 1))
    table = jnp.concatenate(
        [xyz, feat_t, jnp.zeros((B, N, TW - D_CAT), jnp.float32)],
        axis=-1).reshape(B * N, TW)

    scale_f = gf / jnp.sqrt(1.0 + EPS)
    scale_a = ga / jnp.sqrt(1.0 + EPS)
    wf_eff = Wf * scale_f[:, None]
    bf_eff = (bf * scale_f + betaf).reshape(1, OUT)
    wa_eff = Wa * scale_a[:, None]
    ba_eff = (ba * scale_a + betaa).reshape(1, OUT)

    wf_pad = jnp.zeros((TW, OUT), jnp.float32).at[:D_CAT].set(wf_eff.T)
    wa1_pad = jnp.zeros((TW, OUT), jnp.float32).at[:D_CAT].set(wa_eff[:, :D_CAT].T)
    wa2 = wa_eff[:, D_CAT:].T

    f_outs = []
    for b in range(B):
        idxl = _topk_b(xyzp[b], centers_t[b])
        gidx = (idxl + b * N).reshape(NW * GCH, 128)
        g_b = _gather_b(table, gidx).reshape(K, M, TW)
        f_outs.append(_mlp_b(g_b, cpad[b], wf_pad, bf_eff, wa1_pad, wa2, ba_eff))

    f_out = jnp.stack(f_outs)
    f_region = jnp.transpose(f_out, (0, 2, 1))
    return centers, f_region
